# baseline (speedup 1.0000x reference)
import functools

import jax
import jax.numpy as jnp
from jax import lax
from jax.experimental import pallas as pl
from jax.experimental.pallas import tpu as pltpu
from jax.experimental.pallas import tpu_sc as plsc

D = 128
B = 100000
NC = 2
NS = 16
NW = NC * NS
CHUNK = 128
N_CHUNKS = 25
PER_W = N_CHUNKS * CHUNK
B_PAD = NW * PER_W
NBUF = 4
UNROLL = 16
LAST_W = NW - 1
LW_FULL = 6
LW_TAIL = 32


@functools.partial(
    pl.kernel,
    mesh=plsc.VectorSubcoreMesh(core_axis_name="c", subcore_axis_name="s"),
    out_type=jax.ShapeDtypeStruct((B, D), jnp.float32),
    compiler_params=pltpu.CompilerParams(use_tc_tiling_on_sc=False),
    scratch_types=[
        pltpu.VMEM((PER_W,), jnp.int32),
        pltpu.VMEM_SHARED((NS * NBUF * CHUNK, D), jnp.float32),
    ] + [pltpu.SemaphoreType.DMA] * NBUF,
)
def _sc_gather(idx_hbm, table_hbm, out_hbm, idx_v, ring_sh, s0, s1, s2, s3):
    sid = lax.axis_index("s")
    wid = sid * NC + lax.axis_index("c")
    span = wid * PER_W
    pltpu.sync_copy(idx_hbm.at[pl.ds(span, PER_W)], idx_v)

    ring = ring_sh.at[pl.ds(sid * (NBUF * CHUNK), NBUF * CHUNK)]

    sems = (s0, s1, s2, s3)

    def fire(j, b):
        def row_body(k, carry):
            vec = idx_v[pl.ds(j * CHUNK + k * UNROLL, UNROLL)]
            for u in range(UNROLL):
                pltpu.async_copy(
                    table_hbm.at[pl.ds(vec[u], 1)],
                    ring.at[pl.ds(b * CHUNK + k * UNROLL + u, 1)],
                    sems[b])
            return carry

        lax.fori_loop(0, CHUNK // UNROLL, row_body, 0)

    def drain(b):
        pltpu.make_async_copy(
            table_hbm.at[pl.ds(0, CHUNK)],
            ring.at[pl.ds(b * CHUNK, CHUNK)], sems[b]).wait()

    def write(j, b):
        @pl.when(jnp.logical_or(wid < LAST_W, j < LW_FULL))
        def _():
            pltpu.sync_copy(
                ring.at[pl.ds(b * CHUNK, CHUNK)],
                out_hbm.at[pl.ds(span + j * CHUNK, CHUNK)])

        @pl.when(jnp.logical_and(wid == LAST_W, j == LW_FULL))
        def _():
            pltpu.sync_copy(
                ring.at[pl.ds(b * CHUNK, LW_TAIL)],
                out_hbm.at[pl.ds(span + j * CHUNK, LW_TAIL)])

    for b in range(NBUF):
        fire(b, b)

    def body(g, carry):
        for b in range(NBUF):
            j = NBUF * g + b
            drain(b)
            write(j, b)
            fire(j + NBUF, b)
        return carry

    lax.fori_loop(0, 5, body, 0)

    drain(0)
    write(20, 0)
    fire(24, 0)
    for j in range(21, 24):
        b = j % NBUF
        drain(b)
        write(j, b)
    drain(0)
    write(24, 0)


def kernel(source_nodes, source_node_raw_features, timestamps, n_layers,
           node_old_embedding, time_W, time_b):
    idx = source_nodes.astype(jnp.int32)
    idx_pad = jnp.zeros((B_PAD,), jnp.int32).at[:B].set(idx)
    return _sc_gather(idx_pad, node_old_embedding)


# speedup vs baseline: 3.6006x; 3.6006x over previous
import functools

import jax
import jax.numpy as jnp
from jax import lax
from jax.experimental import pallas as pl
from jax.experimental.pallas import tpu as pltpu
from jax.experimental.pallas import tpu_sc as plsc

D = 128
B = 100000
NC = 2
NS = 16
CHUNK = 128
N_GLOBAL = 800
B_PAD = N_GLOBAL * CHUNK
NBUF = 3
FAST_N = 39
SLOW_N = 11
SLOW_BASE = NS * FAST_N
LAST_FULL = (B // CHUNK) - 1
PART = B // CHUNK
PART_ROWS = B - PART * CHUNK


@functools.partial(
    pl.kernel,
    mesh=plsc.VectorSubcoreMesh(core_axis_name="c", subcore_axis_name="s"),
    out_type=jax.ShapeDtypeStruct((B, D), jnp.float32),
    scratch_types=[
        pltpu.VMEM((FAST_N * CHUNK,), jnp.int32),
        pltpu.VMEM((NBUF * CHUNK, D), jnp.float32),
    ] + [pltpu.SemaphoreType.DMA] * NBUF,
)
def _sc_gather(idx_hbm, table_hbm, out_hbm, idx_v, ring, s0, s1, s2):
    cid = lax.axis_index("c")
    sid = lax.axis_index("s")
    sems = (s0, s1, s2)

    def maybe_fire(c, j, b):
        @pl.when(c <= PART)
        def _():
            pltpu.async_copy(
                table_hbm.at[idx_v.at[pl.ds(j * CHUNK, CHUNK)]],
                ring.at[pl.ds(b * CHUNK, CHUNK)],
                sems[b])

    def maybe_drain(c, b):
        @pl.when(c <= PART)
        def _():
            pltpu.make_async_copy(
                table_hbm.at[pl.ds(0, CHUNK)],
                ring.at[pl.ds(b * CHUNK, CHUNK)], sems[b]).wait()

    def write(c, b):
        @pl.when(c <= LAST_FULL)
        def _():
            pltpu.sync_copy(
                ring.at[pl.ds(b * CHUNK, CHUNK)],
                out_hbm.at[pl.ds(c * CHUNK, CHUNK)])

        @pl.when(c == PART)
        def _():
            pltpu.sync_copy(
                ring.at[pl.ds(b * CHUNK, PART_ROWS)],
                out_hbm.at[pl.ds(c * CHUNK, PART_ROWS)])

    def pipeline(base, n):
        pltpu.sync_copy(
            idx_hbm.at[pl.ds(base * CHUNK, n * CHUNK)],
            idx_v.at[pl.ds(0, n * CHUNK)])

        for b in range(NBUF):
            maybe_fire(base + b, b, b)

        niter = (n - NBUF) // NBUF

        def body(g, carry):
            for b in range(NBUF):
                j = NBUF * g + b
                maybe_drain(base + j, b)
                write(base + j, b)
                maybe_fire(base + j + NBUF, j + NBUF, b)
            return carry

        lax.fori_loop(0, niter, body, 0)

        for j in range(NBUF * niter, n):
            b = j % NBUF
            maybe_drain(base + j, b)
            write(base + j, b)
            if j + NBUF <= n - 1:
                maybe_fire(base + j + NBUF, j + NBUF, b)

    @pl.when(cid == 0)
    def _():
        pipeline(sid * FAST_N, FAST_N)

    @pl.when(cid == 1)
    def _():
        pipeline(SLOW_BASE + sid * SLOW_N, SLOW_N)


def kernel(source_nodes, source_node_raw_features, timestamps, n_layers,
           node_old_embedding, time_W, time_b):
    idx = source_nodes.astype(jnp.int32)
    idx_pad = jnp.zeros((B_PAD,), jnp.int32).at[:B].set(idx)
    return _sc_gather(idx_pad, node_old_embedding)


# speedup vs baseline: 3.6872x; 1.0241x over previous
import functools

import jax
import jax.numpy as jnp
from jax import lax
from jax.experimental import pallas as pl
from jax.experimental.pallas import tpu as pltpu
from jax.experimental.pallas import tpu_sc as plsc

D = 128
B = 100000
NC = 2
NS = 16
CHUNK = 128
N_GLOBAL = 800
B_PAD = N_GLOBAL * CHUNK
NBUF = 3
FAST_N = 33
SLOW_N = 17
SLOW_BASE = NS * FAST_N
LAST_FULL = (B // CHUNK) - 1
PART = B // CHUNK
PART_ROWS = B - PART * CHUNK


@functools.partial(
    pl.kernel,
    mesh=plsc.VectorSubcoreMesh(core_axis_name="c", subcore_axis_name="s"),
    out_type=jax.ShapeDtypeStruct((B, D), jnp.float32),
    scratch_types=[
        pltpu.VMEM((FAST_N * CHUNK,), jnp.int32),
        pltpu.VMEM((NBUF * CHUNK, D), jnp.float32),
    ] + [pltpu.SemaphoreType.DMA] * NBUF,
)
def _sc_gather(idx_hbm, table_hbm, out_hbm, idx_v, ring, s0, s1, s2):
    cid = lax.axis_index("c")
    sid = lax.axis_index("s")
    sems = (s0, s1, s2)

    def maybe_fire(c, j, b):
        @pl.when(c <= PART)
        def _():
            pltpu.async_copy(
                table_hbm.at[idx_v.at[pl.ds(j * CHUNK, CHUNK)]],
                ring.at[pl.ds(b * CHUNK, CHUNK)],
                sems[b])

    def maybe_drain(c, b):
        @pl.when(c <= PART)
        def _():
            pltpu.make_async_copy(
                table_hbm.at[pl.ds(0, CHUNK)],
                ring.at[pl.ds(b * CHUNK, CHUNK)], sems[b]).wait()

    def write(c, b):
        @pl.when(c <= LAST_FULL)
        def _():
            pltpu.sync_copy(
                ring.at[pl.ds(b * CHUNK, CHUNK)],
                out_hbm.at[pl.ds(c * CHUNK, CHUNK)])

        @pl.when(c == PART)
        def _():
            pltpu.sync_copy(
                ring.at[pl.ds(b * CHUNK, PART_ROWS)],
                out_hbm.at[pl.ds(c * CHUNK, PART_ROWS)])

    def pipeline(base, n):
        pltpu.sync_copy(
            idx_hbm.at[pl.ds(base * CHUNK, n * CHUNK)],
            idx_v.at[pl.ds(0, n * CHUNK)])

        for b in range(NBUF):
            maybe_fire(base + b, b, b)

        niter = (n - NBUF) // NBUF

        def body(g, carry):
            for b in range(NBUF):
                j = NBUF * g + b
                maybe_drain(base + j, b)
                write(base + j, b)
                maybe_fire(base + j + NBUF, j + NBUF, b)
            return carry

        lax.fori_loop(0, niter, body, 0)

        for j in range(NBUF * niter, n):
            b = j % NBUF
            maybe_drain(base + j, b)
            write(base + j, b)
            if j + NBUF <= n - 1:
                maybe_fire(base + j + NBUF, j + NBUF, b)

    @pl.when(cid == 0)
    def _():
        pipeline(sid * FAST_N, FAST_N)

    @pl.when(cid == 1)
    def _():
        pipeline(SLOW_BASE + sid * SLOW_N, SLOW_N)


def kernel(source_nodes, source_node_raw_features, timestamps, n_layers,
           node_old_embedding, time_W, time_b):
    idx = source_nodes.astype(jnp.int32)
    idx_pad = jnp.zeros((B_PAD,), jnp.int32).at[:B].set(idx)
    return _sc_gather(idx_pad, node_old_embedding)


# speedup vs baseline: 3.8356x; 1.0403x over previous
import functools

import jax
import jax.numpy as jnp
from jax import lax
from jax.experimental import pallas as pl
from jax.experimental.pallas import tpu as pltpu
from jax.experimental.pallas import tpu_sc as plsc

D = 128
B = 100000
NC = 2
NS = 16
CHUNK = 128
N_GLOBAL = 800
B_PAD = N_GLOBAL * CHUNK
NBUF = 3
FAST_N = 31
SLOW_N = 19
SLOW_BASE = NS * FAST_N
LAST_FULL = (B // CHUNK) - 1
PART = B // CHUNK
PART_ROWS = B - PART * CHUNK


@functools.partial(
    pl.kernel,
    mesh=plsc.VectorSubcoreMesh(core_axis_name="c", subcore_axis_name="s"),
    out_type=jax.ShapeDtypeStruct((B, D), jnp.float32),
    scratch_types=[
        pltpu.VMEM((FAST_N * CHUNK,), jnp.int32),
        pltpu.VMEM((NBUF * CHUNK, D), jnp.float32),
    ] + [pltpu.SemaphoreType.DMA] * NBUF,
)
def _sc_gather(idx_hbm, table_hbm, out_hbm, idx_v, ring, s0, s1, s2):
    cid = lax.axis_index("c")
    sid = lax.axis_index("s")
    sems = (s0, s1, s2)

    def maybe_fire(c, j, b):
        @pl.when(c <= PART)
        def _():
            pltpu.async_copy(
                table_hbm.at[idx_v.at[pl.ds(j * CHUNK, CHUNK)]],
                ring.at[pl.ds(b * CHUNK, CHUNK)],
                sems[b])

    def maybe_drain(c, b):
        @pl.when(c <= PART)
        def _():
            pltpu.make_async_copy(
                table_hbm.at[pl.ds(0, CHUNK)],
                ring.at[pl.ds(b * CHUNK, CHUNK)], sems[b]).wait()

    def write(c, b):
        @pl.when(c <= LAST_FULL)
        def _():
            pltpu.sync_copy(
                ring.at[pl.ds(b * CHUNK, CHUNK)],
                out_hbm.at[pl.ds(c * CHUNK, CHUNK)])

        @pl.when(c == PART)
        def _():
            pltpu.sync_copy(
                ring.at[pl.ds(b * CHUNK, PART_ROWS)],
                out_hbm.at[pl.ds(c * CHUNK, PART_ROWS)])

    def pipeline(base, n):
        pltpu.sync_copy(
            idx_hbm.at[pl.ds(base * CHUNK, n * CHUNK)],
            idx_v.at[pl.ds(0, n * CHUNK)])

        for b in range(NBUF):
            maybe_fire(base + b, b, b)

        niter = (n - NBUF) // NBUF

        def body(g, carry):
            for b in range(NBUF):
                j = NBUF * g + b
                maybe_drain(base + j, b)
                write(base + j, b)
                maybe_fire(base + j + NBUF, j + NBUF, b)
            return carry

        lax.fori_loop(0, niter, body, 0)

        for j in range(NBUF * niter, n):
            b = j % NBUF
            maybe_drain(base + j, b)
            write(base + j, b)
            if j + NBUF <= n - 1:
                maybe_fire(base + j + NBUF, j + NBUF, b)

    @pl.when(cid == 0)
    def _():
        pipeline(sid * FAST_N, FAST_N)

    @pl.when(cid == 1)
    def _():
        pipeline(SLOW_BASE + sid * SLOW_N, SLOW_N)


def kernel(source_nodes, source_node_raw_features, timestamps, n_layers,
           node_old_embedding, time_W, time_b):
    idx = source_nodes.astype(jnp.int32)
    idx_pad = jnp.zeros((B_PAD,), jnp.int32).at[:B].set(idx)
    return _sc_gather(idx_pad, node_old_embedding)


# speedup vs baseline: 3.9527x; 1.0305x over previous
import functools

import jax
import jax.numpy as jnp
from jax import lax
from jax.experimental import pallas as pl
from jax.experimental.pallas import tpu as pltpu
from jax.experimental.pallas import tpu_sc as plsc

D = 128
B = 100000
NC = 2
NS = 16
CHUNK = 128
N_GLOBAL = 800
B_PAD = N_GLOBAL * CHUNK
NBUF = 3
FAST_N = 31
SLOW_N = 19
SLOW_BASE = NS * FAST_N
LAST_FULL = (B // CHUNK) - 1
PART = B // CHUNK
PART_ROWS = B - PART * CHUNK


@functools.partial(
    pl.kernel,
    mesh=plsc.VectorSubcoreMesh(core_axis_name="c", subcore_axis_name="s"),
    out_type=jax.ShapeDtypeStruct((B, D), jnp.float32),
    scratch_types=[
        pltpu.VMEM((FAST_N * CHUNK,), jnp.int32),
        pltpu.VMEM((NBUF * CHUNK, D), jnp.float32),
    ] + [pltpu.SemaphoreType.DMA] * NBUF,
)
def _sc_gather(idx_hbm, table_hbm, out_hbm, idx_v, ring, s0, s1, s2):
    cid = lax.axis_index("c")
    sid = lax.axis_index("s")
    sems = (s0, s1, s2)

    def maybe_fire(c, j, b):
        @pl.when(c <= LAST_FULL)
        def _():
            pltpu.async_copy(
                table_hbm.at[idx_v.at[pl.ds(j * CHUNK, CHUNK)]],
                ring.at[pl.ds(b * CHUNK, CHUNK)],
                sems[b])

        @pl.when(c == PART)
        def _():
            pltpu.async_copy(
                table_hbm.at[idx_v.at[pl.ds(j * CHUNK, PART_ROWS)]],
                ring.at[pl.ds(b * CHUNK, PART_ROWS)],
                sems[b])

    def maybe_drain(c, b):
        @pl.when(c <= LAST_FULL)
        def _():
            pltpu.make_async_copy(
                table_hbm.at[pl.ds(0, CHUNK)],
                ring.at[pl.ds(b * CHUNK, CHUNK)], sems[b]).wait()

        @pl.when(c == PART)
        def _():
            pltpu.make_async_copy(
                table_hbm.at[pl.ds(0, PART_ROWS)],
                ring.at[pl.ds(b * CHUNK, PART_ROWS)], sems[b]).wait()

    def write(c, b):
        @pl.when(c <= LAST_FULL)
        def _():
            pltpu.sync_copy(
                ring.at[pl.ds(b * CHUNK, CHUNK)],
                out_hbm.at[pl.ds(c * CHUNK, CHUNK)])

        @pl.when(c == PART)
        def _():
            pltpu.sync_copy(
                ring.at[pl.ds(b * CHUNK, PART_ROWS)],
                out_hbm.at[pl.ds(c * CHUNK, PART_ROWS)])

    def pipeline(base, n):
        @pl.when(base + n <= PART + 1)
        def _():
            pltpu.sync_copy(
                idx_hbm.at[pl.ds(base * CHUNK, n * CHUNK)],
                idx_v.at[pl.ds(0, n * CHUNK)])

        @pl.when(base + n > PART + 1)
        def _():
            pltpu.sync_copy(
                idx_hbm.at[pl.ds(base * CHUNK, PART_ROWS)],
                idx_v.at[pl.ds(0, PART_ROWS)])

        for b in range(NBUF):
            maybe_fire(base + b, b, b)

        niter = (n - NBUF) // NBUF

        def body(g, carry):
            for b in range(NBUF):
                j = NBUF * g + b
                maybe_drain(base + j, b)
                write(base + j, b)
                maybe_fire(base + j + NBUF, j + NBUF, b)
            return carry

        lax.fori_loop(0, niter, body, 0)

        for j in range(NBUF * niter, n):
            b = j % NBUF
            maybe_drain(base + j, b)
            write(base + j, b)
            if j + NBUF <= n - 1:
                maybe_fire(base + j + NBUF, j + NBUF, b)

    @pl.when(cid == 0)
    def _():
        pipeline(sid * FAST_N, FAST_N)

    @pl.when(cid == 1)
    def _():
        pipeline(SLOW_BASE + sid * SLOW_N, SLOW_N)


def kernel(source_nodes, source_node_raw_features, timestamps, n_layers,
           node_old_embedding, time_W, time_b):
    return _sc_gather(source_nodes.astype(jnp.int32), node_old_embedding)
